# baseline (speedup 1.0000x reference)
import math

import jax
import jax.numpy as jnp
from jax.experimental import pallas as pl
from jax.experimental.pallas import tpu as pltpu

B = 4
L = 2048
H = 16
C = 64
D = H * C
S = 256
NS = L // S
TOPK = int(2 * math.log(L))
KPAD = 16

TILE = 256
NT = L // TILE
DC = 512
NDC = D // DC


def _corr_kernel(k_ref, q_ref, v_ref, idx_ref, w_ref, v2_ref, acc_scr, r_scr):
    b = pl.program_id(0)
    s = pl.program_id(1)

    v2_ref[0, 0] = v_ref[0]
    v2_ref[0, 1] = v_ref[0]

    m = jax.lax.dot_general(
        k_ref[0], q_ref[0], (((1,), (1,)), ((), ())),
        preferred_element_type=jnp.float32)
    sheared = pltpu.roll(m, 0, 1, stride=1, stride_axis=0)
    r = jnp.sum(sheared, axis=0, keepdims=True)
    vb = pltpu.roll(r, (2 * L - (s * S + S - 1)) % L, 1)

    @pl.when(s == 0)
    def _init_acc():
        acc_scr[...] = vb

    @pl.when(s > 0)
    def _add_acc():
        acc_scr[...] = acc_scr[...] + vb

    @pl.when(s == NS - 1)
    def _finish_batch():
        r_scr[pl.ds(b, 1), :] = acc_scr[...] * (1.0 / D)

    @pl.when((b == B - 1) & (s == NS - 1))
    def _topk():
        rfull = r_scr[:]
        u = jnp.sum(rfull, axis=0, keepdims=True)
        lane = jax.lax.broadcasted_iota(jnp.int32, (1, L), 1)
        laneb = jax.lax.broadcasted_iota(jnp.int32, (B, L), 1)
        cols = []
        idxs = []
        for _ in range(TOPK):
            mx = jnp.max(u)
            idx = jnp.min(jnp.where(u == mx, lane, L))
            idxs.append(idx)
            cols.append(jnp.sum(jnp.where(laneb == idx, rfull, 0.0),
                                axis=1, keepdims=True))
            u = jnp.where(lane == idx, -jnp.inf, u)
        wmat = jnp.concatenate(
            cols + [jnp.full((B, KPAD - TOPK), -jnp.inf, jnp.float32)], axis=1)
        wmax = jnp.max(wmat, axis=1, keepdims=True)
        we = jnp.exp(wmat - wmax)
        w_ref[...] = we / jnp.sum(we, axis=1, keepdims=True)

        klane = jax.lax.broadcasted_iota(jnp.int32, (1, KPAD), 1)
        ivec = jnp.zeros((1, KPAD), jnp.int32)
        for i in range(TOPK):
            ivec = ivec + jnp.where(klane == i, idxs[i], 0)
        idx_ref[...] = ivec


def _agg_kernel(idx_ref, w_ref, v2_ref, o_ref):
    b = pl.program_id(0)
    t = pl.program_id(1)
    base = t * TILE
    acc = jnp.zeros((TILE, 8, 128), jnp.float32)
    for i in range(TOPK):
        acc = acc + w_ref[b, i] * v2_ref[0, pl.ds(base + idx_ref[0, i], TILE)]
    o_ref[0] = acc


def kernel(query, key, value):
    q3 = query.reshape(B, L, D)
    kf = key.reshape(B, L, D)[:, ::-1, :]
    v4 = value.reshape(B, L, 8, 128)

    idx, w, v2d = pl.pallas_call(
        _corr_kernel,
        grid=(B, NS),
        in_specs=[
            pl.BlockSpec((1, S, D), lambda b, s: (b, NS - 1 - s, 0)),
            pl.BlockSpec((1, L, D), lambda b, s: (b, 0, 0)),
            pl.BlockSpec((1, S, 8, 128), lambda b, s: (b, s, 0, 0)),
        ],
        out_specs=[
            pl.BlockSpec((1, KPAD), lambda b, s: (0, 0)),
            pl.BlockSpec((B, KPAD), lambda b, s: (0, 0)),
            pl.BlockSpec((1, 2, S, 8, 128), lambda b, s: (b, 0, s, 0, 0)),
        ],
        out_shape=[
            jax.ShapeDtypeStruct((1, KPAD), jnp.int32),
            jax.ShapeDtypeStruct((B, KPAD), jnp.float32),
            jax.ShapeDtypeStruct((B, 2, L, 8, 128), jnp.float32),
        ],
        scratch_shapes=[
            pltpu.VMEM((1, L), jnp.float32),
            pltpu.VMEM((B, L), jnp.float32),
        ],
    )(kf, q3, v4)

    v2 = v2d.reshape(B, 2 * L, 8, 128)

    out = pl.pallas_call(
        _agg_kernel,
        grid=(B, NT),
        in_specs=[
            pl.BlockSpec(memory_space=pltpu.SMEM),
            pl.BlockSpec(memory_space=pltpu.SMEM),
            pl.BlockSpec((1, 2 * L, 8, 128), lambda b, t: (b, 0, 0, 0)),
        ],
        out_specs=pl.BlockSpec((1, TILE, 8, 128), lambda b, t: (b, t, 0, 0)),
        out_shape=jax.ShapeDtypeStruct((B, L, 8, 128), jnp.float32),
    )(idx, w, v2)

    return out.reshape(B, L, H, C)


# speedup vs baseline: 1.9073x; 1.9073x over previous
import math

import jax
import jax.numpy as jnp
from jax.experimental import pallas as pl
from jax.experimental.pallas import tpu as pltpu

B = 4
L = 2048
H = 16
C = 64
D = H * C
S = 256
NS = L // S
TOPK = int(2 * math.log(L))
KPAD = 16

TILE = 256
NT = L // TILE
DC = 512
NDC = D // DC


def _corr_kernel(k_ref, q_ref, v_ref, idx_ref, w_ref, v2_ref, acc_scr, r_scr):
    b = pl.program_id(0)
    s = pl.program_id(1)

    v2_ref[0, 0] = v_ref[0]
    v2_ref[0, 1] = v_ref[0]

    ia = jax.lax.broadcasted_iota(jnp.int32, (S, S), 0)
    ib = jax.lax.broadcasted_iota(jnp.int32, (S, S), 1)
    jmat = jnp.where(ib == (S - 1) - ia, 1.0, 0.0)
    kflip = jax.lax.dot_general(
        jmat, k_ref[0], (((1,), (0,)), ((), ())),
        preferred_element_type=jnp.float32)
    m = jax.lax.dot_general(
        kflip, q_ref[0], (((1,), (1,)), ((), ())),
        preferred_element_type=jnp.float32)
    sheared = pltpu.roll(m, 0, 1, stride=1, stride_axis=0)
    r = jnp.sum(sheared, axis=0, keepdims=True)
    vb = pltpu.roll(r, (2 * L - (s * S + S - 1)) % L, 1)

    @pl.when(s == 0)
    def _init_acc():
        acc_scr[...] = vb

    @pl.when(s > 0)
    def _add_acc():
        acc_scr[...] = acc_scr[...] + vb

    @pl.when(s == NS - 1)
    def _finish_batch():
        r_scr[pl.ds(b, 1), :] = acc_scr[...] * (1.0 / D)

    @pl.when((b == B - 1) & (s == NS - 1))
    def _topk():
        rfull = r_scr[:]
        u = jnp.sum(rfull, axis=0, keepdims=True)
        lane = jax.lax.broadcasted_iota(jnp.int32, (1, L), 1)
        laneb = jax.lax.broadcasted_iota(jnp.int32, (B, L), 1)
        cols = []
        idxs = []
        for _ in range(TOPK):
            mx = jnp.max(u)
            idx = jnp.min(jnp.where(u == mx, lane, L))
            idxs.append(idx)
            cols.append(jnp.sum(jnp.where(laneb == idx, rfull, 0.0),
                                axis=1, keepdims=True))
            u = jnp.where(lane == idx, -jnp.inf, u)
        wmat = jnp.concatenate(
            cols + [jnp.full((B, KPAD - TOPK), -jnp.inf, jnp.float32)], axis=1)
        wmax = jnp.max(wmat, axis=1, keepdims=True)
        we = jnp.exp(wmat - wmax)
        w_ref[...] = we / jnp.sum(we, axis=1, keepdims=True)

        klane = jax.lax.broadcasted_iota(jnp.int32, (1, KPAD), 1)
        ivec = jnp.zeros((1, KPAD), jnp.int32)
        for i in range(TOPK):
            ivec = ivec + jnp.where(klane == i, idxs[i], 0)
        idx_ref[...] = ivec


def _agg_kernel(idx_ref, w_ref, v2_ref, o_ref):
    b = pl.program_id(0)
    t = pl.program_id(1)
    base = t * TILE
    acc = jnp.zeros((TILE, 8, 128), jnp.float32)
    for i in range(TOPK):
        acc = acc + w_ref[b, i] * v2_ref[0, pl.ds(base + idx_ref[0, i], TILE)]
    o_ref[0] = acc


def kernel(query, key, value):
    q3 = query.reshape(B, L, D)
    k3 = key.reshape(B, L, D)
    v4 = value.reshape(B, L, 8, 128)

    idx, w, v2d = pl.pallas_call(
        _corr_kernel,
        grid=(B, NS),
        in_specs=[
            pl.BlockSpec((1, S, D), lambda b, s: (b, s, 0)),
            pl.BlockSpec((1, L, D), lambda b, s: (b, 0, 0)),
            pl.BlockSpec((1, S, 8, 128), lambda b, s: (b, s, 0, 0)),
        ],
        out_specs=[
            pl.BlockSpec((1, KPAD), lambda b, s: (0, 0)),
            pl.BlockSpec((B, KPAD), lambda b, s: (0, 0)),
            pl.BlockSpec((1, 2, S, 8, 128), lambda b, s: (b, 0, s, 0, 0)),
        ],
        out_shape=[
            jax.ShapeDtypeStruct((1, KPAD), jnp.int32),
            jax.ShapeDtypeStruct((B, KPAD), jnp.float32),
            jax.ShapeDtypeStruct((B, 2, L, 8, 128), jnp.float32),
        ],
        scratch_shapes=[
            pltpu.VMEM((1, L), jnp.float32),
            pltpu.VMEM((B, L), jnp.float32),
        ],
    )(k3, q3, v4)

    v2 = v2d.reshape(B, 2 * L, 8, 128)

    out = pl.pallas_call(
        _agg_kernel,
        grid=(B, NT),
        in_specs=[
            pl.BlockSpec(memory_space=pltpu.SMEM),
            pl.BlockSpec(memory_space=pltpu.SMEM),
            pl.BlockSpec((1, 2 * L, 8, 128), lambda b, t: (b, 0, 0, 0)),
        ],
        out_specs=pl.BlockSpec((1, TILE, 8, 128), lambda b, t: (b, t, 0, 0)),
        out_shape=jax.ShapeDtypeStruct((B, L, 8, 128), jnp.float32),
    )(idx, w, v2)

    return out.reshape(B, L, H, C)
